# Initial kernel scaffold; baseline (speedup 1.0000x reference)
import functools

import jax
import jax.numpy as jnp
from jax import lax
from jax.experimental import pallas as pl
from jax.experimental.pallas import tpu as pltpu
from jax.experimental.pallas import tpu_sc as plsc

N = 50000
E = 800000
H = 64
HH = 32

NP = 50176
EP = 819200
NTILES = 16
ROWS_PER_TILE = NP // NTILES
EROWS = EP // 128
EROWS_PER_TILE = EROWS // NTILES
BLK = 8
NBLK = EROWS_PER_TILE // BLK
DEG_EROWS_PER_TILE = EROWS // (2 * NTILES)
DEG_NBLK = DEG_EROWS_PER_TILE // BLK

BN = 512
GRID = NP // BN

_SC_MESH = plsc.VectorSubcoreMesh(core_axis_name="c", subcore_axis_name="s")


def _deg_body(dstr, ones_h, zeros_h, out0, out1, ones_v, idx_d, acc, sem):
    c = lax.axis_index("c")
    s = lax.axis_index("s")
    rbase = s * ROWS_PER_TILE
    pltpu.sync_copy(zeros_h.at[pl.ds(rbase, ROWS_PER_TILE)],
                    acc.at[pl.ds(rbase, ROWS_PER_TILE)])
    pltpu.sync_copy(ones_h, ones_v)
    plsc.subcore_barrier()

    ebase = c * (EROWS // 2) + s * DEG_EROWS_PER_TILE

    def blk(b, carry):
        r0 = ebase + b * BLK
        pltpu.sync_copy(dstr.at[pl.ds(r0, BLK)], idx_d)
        for j in range(BLK):
            pltpu.sync_copy(ones_v, acc.at[idx_d.at[j]], add=True)
        return carry

    lax.fori_loop(0, DEG_NBLK, blk, 0)
    plsc.subcore_barrier()

    @pl.when(c == 0)
    def _():
        pltpu.sync_copy(acc.at[pl.ds(rbase, ROWS_PER_TILE)],
                        out0.at[pl.ds(rbase, ROWS_PER_TILE)])

    @pl.when(c == 1)
    def _():
        pltpu.sync_copy(acc.at[pl.ds(rbase, ROWS_PER_TILE)],
                        out1.at[pl.ds(rbase, ROWS_PER_TILE)])


_sc_degree = pl.kernel(
    _deg_body,
    out_type=(jax.ShapeDtypeStruct((NP, 8), jnp.float32),
              jax.ShapeDtypeStruct((NP, 8), jnp.float32)),
    mesh=_SC_MESH,
    scratch_types=[
        pltpu.VMEM((128, 8), jnp.float32),
        pltpu.VMEM((BLK, 128), jnp.int32),
        pltpu.VMEM_SHARED((NP, 8), jnp.float32),
        pltpu.SemaphoreType.DMA,
    ],
)


def _spmm_body(t0, t1, srcr, dstr, out0, out1, idx_s, idx_d, rows, acc, sem):
    c = lax.axis_index("c")
    s = lax.axis_index("s")
    rbase = s * ROWS_PER_TILE

    @pl.when(c == 0)
    def _():
        pltpu.sync_copy(t0.at[pl.ds(rbase, ROWS_PER_TILE)],
                        acc.at[pl.ds(rbase, ROWS_PER_TILE)])

    @pl.when(c == 1)
    def _():
        pltpu.sync_copy(t1.at[pl.ds(rbase, ROWS_PER_TILE)],
                        acc.at[pl.ds(rbase, ROWS_PER_TILE)])

    plsc.subcore_barrier()

    ebase = s * EROWS_PER_TILE

    def blk(b, carry):
        r0 = ebase + b * BLK
        pltpu.sync_copy(srcr.at[pl.ds(r0, BLK)], idx_s)
        pltpu.sync_copy(dstr.at[pl.ds(r0, BLK)], idx_d)

        @pl.when(c == 0)
        def _():
            cps = [pltpu.async_copy(t0.at[idx_s.at[j]], rows.at[j], sem)
                   for j in range(BLK)]
            for cp in cps:
                cp.wait()

        @pl.when(c == 1)
        def _():
            cps = [pltpu.async_copy(t1.at[idx_s.at[j]], rows.at[j], sem)
                   for j in range(BLK)]
            for cp in cps:
                cp.wait()

        for j in range(BLK):
            pltpu.sync_copy(rows.at[j], acc.at[idx_d.at[j]], add=True)
        return carry

    lax.fori_loop(0, NBLK, blk, 0)
    plsc.subcore_barrier()

    @pl.when(c == 0)
    def _():
        pltpu.sync_copy(acc.at[pl.ds(rbase, ROWS_PER_TILE)],
                        out0.at[pl.ds(rbase, ROWS_PER_TILE)])

    @pl.when(c == 1)
    def _():
        pltpu.sync_copy(acc.at[pl.ds(rbase, ROWS_PER_TILE)],
                        out1.at[pl.ds(rbase, ROWS_PER_TILE)])


_sc_spmm = pl.kernel(
    _spmm_body,
    out_type=(jax.ShapeDtypeStruct((NP, HH), jnp.float32),
              jax.ShapeDtypeStruct((NP, HH), jnp.float32)),
    mesh=_SC_MESH,
    scratch_types=[
        pltpu.VMEM((BLK, 128), jnp.int32),
        pltpu.VMEM((BLK, 128), jnp.int32),
        pltpu.VMEM((BLK, 128, HH), jnp.float32),
        pltpu.VMEM_SHARED((NP, HH), jnp.float32),
        pltpu.SemaphoreType.DMA,
    ],
)


def _row_spec(w):
    return pl.BlockSpec((BN, w), lambda i: (i, 0))


def _full_spec(shape):
    return pl.BlockSpec(shape, lambda i: tuple(0 for _ in shape))


def _relu(v):
    return jnp.maximum(v, 0.0)


def _enc_body(x, pin, d0, d1, W_se, b_se, W_pe1, b_pe1, W_pe2, b_pe2,
              W_c1, W_pc1, dinv_o, ts0, ts1, tp0, tp1):
    cnt = d0[:, :1] + d1[:, :1]
    dinv = lax.rsqrt(cnt + 1.0)
    s0 = _relu(x[:, 0:1] * W_se[0:1, :] + x[:, 1:2] * W_se[1:2, :] + b_se[...])
    p = _relu(pin[:, 0:1] * W_pe1[0:1, :] + pin[:, 1:2] * W_pe1[1:2, :]
              + b_pe1[...])
    p0 = _relu(jnp.dot(p, W_pe2[...], preferred_element_type=jnp.float32)
               + b_pe2[...])
    hs = dinv * jnp.dot(s0, W_c1[...], preferred_element_type=jnp.float32)
    hp = dinv * jnp.dot(p0, W_pc1[...], preferred_element_type=jnp.float32)
    dinv_o[...] = dinv
    ts0[...] = hs[:, :HH]
    ts1[...] = hs[:, HH:]
    tp0[...] = hp[:, :HH]
    tp1[...] = hp[:, HH:]


def _tc_enc(x, pin, d0, d1, W_se, b_se, W_pe1, b_pe1, W_pe2, b_pe2,
            W_c1, W_pc1):
    return pl.pallas_call(
        _enc_body,
        grid=(GRID,),
        in_specs=[_row_spec(2), _row_spec(2), _row_spec(8), _row_spec(8),
                  _full_spec((2, H)), _full_spec((1, H)),
                  _full_spec((2, H)), _full_spec((1, H)),
                  _full_spec((H, H)), _full_spec((1, H)),
                  _full_spec((H, H)), _full_spec((H, H))],
        out_specs=[_row_spec(1), _row_spec(HH), _row_spec(HH),
                   _row_spec(HH), _row_spec(HH)],
        out_shape=[jax.ShapeDtypeStruct((NP, 1), jnp.float32)]
        + [jax.ShapeDtypeStruct((NP, HH), jnp.float32)] * 4,
    )(x, pin, d0, d1, W_se, b_se, W_pe1, b_pe1, W_pe2, b_pe2, W_c1, W_pc1)


def _round_body(Ss0, Ss1, Sp0, Sp1, dinv, b_s, b_p, W_s, W_p,
                ts0, ts1, tp0, tp1):
    dv = dinv[...]
    s = _relu(dv * jnp.concatenate([Ss0[...], Ss1[...]], axis=1) + b_s[...])
    p = _relu(dv * jnp.concatenate([Sp0[...], Sp1[...]], axis=1) + b_p[...])
    hs = dv * jnp.dot(s, W_s[...], preferred_element_type=jnp.float32)
    hp = dv * jnp.dot(p, W_p[...], preferred_element_type=jnp.float32)
    ts0[...] = hs[:, :HH]
    ts1[...] = hs[:, HH:]
    tp0[...] = hp[:, :HH]
    tp1[...] = hp[:, HH:]


def _tc_round(Ss0, Ss1, Sp0, Sp1, dinv, b_s, b_p, W_s, W_p):
    return pl.pallas_call(
        _round_body,
        grid=(GRID,),
        in_specs=[_row_spec(HH)] * 4 + [_row_spec(1),
                  _full_spec((1, H)), _full_spec((1, H)),
                  _full_spec((H, H)), _full_spec((H, H))],
        out_specs=[_row_spec(HH)] * 4,
        out_shape=[jax.ShapeDtypeStruct((NP, HH), jnp.float32)] * 4,
    )(Ss0, Ss1, Sp0, Sp1, dinv, b_s, b_p, W_s, W_p)


def _mix_body(Ss0, Ss1, Sp0, Sp1, dinv, b_s, b_p, W_t, W_b, tc0, tc1):
    dv = dinv[...]
    s = _relu(dv * jnp.concatenate([Ss0[...], Ss1[...]], axis=1) + b_s[...])
    p = _relu(dv * jnp.concatenate([Sp0[...], Sp1[...]], axis=1) + b_p[...])
    hc = dv * (jnp.dot(s, W_t[...], preferred_element_type=jnp.float32)
               + jnp.dot(p, W_b[...], preferred_element_type=jnp.float32))
    tc0[...] = hc[:, :HH]
    tc1[...] = hc[:, HH:]


def _tc_mix(Ss0, Ss1, Sp0, Sp1, dinv, b_s, b_p, W_t, W_b):
    return pl.pallas_call(
        _mix_body,
        grid=(GRID,),
        in_specs=[_row_spec(HH)] * 4 + [_row_spec(1),
                  _full_spec((1, H)), _full_spec((1, H)),
                  _full_spec((H, H)), _full_spec((H, H))],
        out_specs=[_row_spec(HH)] * 2,
        out_shape=[jax.ShapeDtypeStruct((NP, HH), jnp.float32)] * 2,
    )(Ss0, Ss1, Sp0, Sp1, dinv, b_s, b_p, W_t, W_b)


def _fin_body(Sc0, Sc1, dinv, b_cc, W_d1, b_d1, W_d2, b_d2, W_d3, b_d3, out):
    dv = dinv[...]
    c = _relu(dv * jnp.concatenate([Sc0[...], Sc1[...]], axis=1) + b_cc[...])
    d = _relu(jnp.dot(c, W_d1[...], preferred_element_type=jnp.float32)
              + b_d1[...])
    d = _relu(jnp.dot(d, W_d2[...], preferred_element_type=jnp.float32)
              + b_d2[...])
    d3 = jnp.sum(d * W_d3[...], axis=1, keepdims=True) + b_d3[...]
    out[...] = 1.0 / (1.0 + jnp.exp(-d3))


def _tc_fin(Sc0, Sc1, dinv, b_cc, W_d1, b_d1, W_d2, b_d2, W_d3, b_d3):
    return pl.pallas_call(
        _fin_body,
        grid=(GRID,),
        in_specs=[_row_spec(HH)] * 2 + [_row_spec(1),
                  _full_spec((1, H)),
                  _full_spec((H, H)), _full_spec((1, H)),
                  _full_spec((H, HH)), _full_spec((1, HH)),
                  _full_spec((1, HH)), _full_spec((1, 1))],
        out_specs=[_row_spec(1)],
        out_shape=[jax.ShapeDtypeStruct((NP, 1), jnp.float32)],
    )(Sc0, Sc1, dinv, b_cc, W_d1, b_d1, W_d2, b_d2, W_d3, b_d3)[0]


@jax.jit
def _run(x, true_alpha_t, true_torque_t, edge_index,
         W_se, b_se, W_pe1, b_pe1, W_pe2, b_pe2,
         W_c1, b_c1, W_c2, b_c2, W_pc1, b_pc1, W_pc2, b_pc2,
         W_cc, b_cc, W_d1, b_d1, W_d2, b_d2, W_d3, b_d3):
    f32 = jnp.float32
    xp = jnp.zeros((NP, 2), f32).at[:N].set(x)
    pin = jnp.zeros((NP, 2), f32).at[:N, 0].set(true_alpha_t[:, 0])
    pin = pin.at[:N, 1].set(true_torque_t[:, 0])
    pad = jnp.full((EP - E,), N, jnp.int32)
    srcr = jnp.concatenate([edge_index[0], pad]).reshape(EROWS, 128)
    dstr = jnp.concatenate([edge_index[1], pad]).reshape(EROWS, 128)
    ones8 = jnp.ones((128, 8), f32)
    zeros8 = jnp.zeros((NP, 8), f32)

    def row(b):
        return b.reshape(1, -1)

    d0, d1 = _sc_degree(dstr, ones8, zeros8)

    dinv, ts0, ts1, tp0, tp1 = _tc_enc(
        xp, pin, d0, d1, W_se, row(b_se), W_pe1, row(b_pe1), W_pe2,
        row(b_pe2), W_c1, W_pc1)

    Ss0, Ss1 = _sc_spmm(ts0, ts1, srcr, dstr)
    Sp0, Sp1 = _sc_spmm(tp0, tp1, srcr, dstr)
    ts0, ts1, tp0, tp1 = _tc_round(Ss0, Ss1, Sp0, Sp1, dinv,
                                   row(b_c1), row(b_pc1), W_c2, W_pc2)

    Ss0, Ss1 = _sc_spmm(ts0, ts1, srcr, dstr)
    Sp0, Sp1 = _sc_spmm(tp0, tp1, srcr, dstr)
    tc0, tc1 = _tc_mix(Ss0, Ss1, Sp0, Sp1, dinv, row(b_c2), row(b_pc2),
                       W_cc[:H], W_cc[H:])

    Sc0, Sc1 = _sc_spmm(tc0, tc1, srcr, dstr)
    out = _tc_fin(Sc0, Sc1, dinv, row(b_cc), W_d1, row(b_d1),
                  W_d2, row(b_d2), W_d3.reshape(1, HH), b_d3.reshape(1, 1))
    return out[:N]


def kernel(x, true_alpha_t, true_torque_t, edge_index,
           W_se, b_se, W_pe1, b_pe1, W_pe2, b_pe2,
           W_c1, b_c1, W_c2, b_c2, W_pc1, b_pc1, W_pc2, b_pc2,
           W_cc, b_cc, W_d1, b_d1, W_d2, b_d2, W_d3, b_d3):
    return _run(x, true_alpha_t, true_torque_t, edge_index,
                W_se, b_se, W_pe1, b_pe1, W_pe2, b_pe2,
                W_c1, b_c1, W_c2, b_c2, W_pc1, b_pc1, W_pc2, b_pc2,
                W_cc, b_cc, W_d1, b_d1, W_d2, b_d2, W_d3, b_d3)


# speedup vs baseline: 8.9734x; 8.9734x over previous
import functools

import jax
import jax.numpy as jnp
from jax import lax
from jax.experimental import pallas as pl
from jax.experimental.pallas import tpu as pltpu
from jax.experimental.pallas import tpu_sc as plsc

N = 50000
E = 800000
H = 64
HH = 32

NP = 50176
EP = 819200
NTILES = 16
ROWS_PER_TILE = NP // NTILES
EROWS = EP // 128
EROWS_PER_TILE = EROWS // NTILES
BLK = 4
NBLK = EROWS_PER_TILE // BLK

BN = 512
GRID = NP // BN

_SC_MESH = plsc.VectorSubcoreMesh(core_axis_name="c", subcore_axis_name="s")
_SC_PARAMS = pltpu.CompilerParams(use_tc_tiling_on_sc=False)


def _spmm_body(t0, t1, srcr, dstr, out0, out1, idx_s, idx_d, rows, acc, sem):
    c = lax.axis_index("c")
    s = lax.axis_index("s")
    rbase = s * ROWS_PER_TILE

    @pl.when(c == 0)
    def _():
        pltpu.sync_copy(t0.at[pl.ds(rbase, ROWS_PER_TILE)],
                        acc.at[pl.ds(rbase, ROWS_PER_TILE)])

    @pl.when(c == 1)
    def _():
        pltpu.sync_copy(t1.at[pl.ds(rbase, ROWS_PER_TILE)],
                        acc.at[pl.ds(rbase, ROWS_PER_TILE)])

    plsc.subcore_barrier()

    ebase = s * EROWS_PER_TILE

    def blk(b, carry):
        r0 = ebase + b * BLK
        pltpu.sync_copy(srcr.at[pl.ds(r0, BLK)], idx_s)
        pltpu.sync_copy(dstr.at[pl.ds(r0, BLK)], idx_d)

        @pl.when(c == 0)
        def _():
            cps = [pltpu.async_copy(t0.at[idx_s.at[j]], rows.at[j], sem)
                   for j in range(BLK)]
            for cp in cps:
                cp.wait()

        @pl.when(c == 1)
        def _():
            cps = [pltpu.async_copy(t1.at[idx_s.at[j]], rows.at[j], sem)
                   for j in range(BLK)]
            for cp in cps:
                cp.wait()

        for j in range(BLK):
            pltpu.sync_copy(rows.at[j], acc.at[idx_d.at[j]], add=True)
        return carry

    lax.fori_loop(0, NBLK, blk, 0)
    plsc.subcore_barrier()

    @pl.when(c == 0)
    def _():
        pltpu.sync_copy(acc.at[pl.ds(rbase, ROWS_PER_TILE)],
                        out0.at[pl.ds(rbase, ROWS_PER_TILE)])

    @pl.when(c == 1)
    def _():
        pltpu.sync_copy(acc.at[pl.ds(rbase, ROWS_PER_TILE)],
                        out1.at[pl.ds(rbase, ROWS_PER_TILE)])


_sc_spmm = pl.kernel(
    _spmm_body,
    out_type=(jax.ShapeDtypeStruct((NP, HH), jnp.float32),
              jax.ShapeDtypeStruct((NP, HH), jnp.float32)),
    mesh=_SC_MESH,
    scratch_types=[
        pltpu.VMEM((BLK, 128), jnp.int32),
        pltpu.VMEM((BLK, 128), jnp.int32),
        pltpu.VMEM((BLK, 128, HH), jnp.float32),
        pltpu.VMEM_SHARED((NP, HH), jnp.float32),
        pltpu.SemaphoreType.DMA,
    ],
    compiler_params=_SC_PARAMS,
)


def _row_spec(w):
    return pl.BlockSpec((BN, w), lambda i: (i, 0))


def _full_spec(shape):
    return pl.BlockSpec(shape, lambda i: tuple(0 for _ in shape))


def _relu(v):
    return jnp.maximum(v, 0.0)


def _enc_body(x, pin, d0, W_se, b_se, W_pe1, b_pe1, W_pe2, b_pe2,
              W_c1, W_pc1, dinv_o, ts0, ts1, tp0, tp1):
    dinv = lax.rsqrt(d0[:, :1])
    s0 = _relu(x[:, 0:1] * W_se[0:1, :] + x[:, 1:2] * W_se[1:2, :] + b_se[...])
    p = _relu(pin[:, 0:1] * W_pe1[0:1, :] + pin[:, 1:2] * W_pe1[1:2, :]
              + b_pe1[...])
    p0 = _relu(jnp.dot(p, W_pe2[...], preferred_element_type=jnp.float32)
               + b_pe2[...])
    hs = dinv * jnp.dot(s0, W_c1[...], preferred_element_type=jnp.float32)
    hp = dinv * jnp.dot(p0, W_pc1[...], preferred_element_type=jnp.float32)
    dinv_o[...] = dinv
    ts0[...] = hs[:, :HH]
    ts1[...] = hs[:, HH:]
    tp0[...] = hp[:, :HH]
    tp1[...] = hp[:, HH:]


def _tc_enc(x, pin, d0, W_se, b_se, W_pe1, b_pe1, W_pe2, b_pe2,
            W_c1, W_pc1):
    return pl.pallas_call(
        _enc_body,
        grid=(GRID,),
        in_specs=[_row_spec(2), _row_spec(2), _row_spec(HH),
                  _full_spec((2, H)), _full_spec((1, H)),
                  _full_spec((2, H)), _full_spec((1, H)),
                  _full_spec((H, H)), _full_spec((1, H)),
                  _full_spec((H, H)), _full_spec((H, H))],
        out_specs=[_row_spec(1), _row_spec(HH), _row_spec(HH),
                   _row_spec(HH), _row_spec(HH)],
        out_shape=[jax.ShapeDtypeStruct((NP, 1), jnp.float32)]
        + [jax.ShapeDtypeStruct((NP, HH), jnp.float32)] * 4,
    )(x, pin, d0, W_se, b_se, W_pe1, b_pe1, W_pe2, b_pe2, W_c1, W_pc1)


def _round_body(Ss0, Ss1, Sp0, Sp1, dinv, b_s, b_p, W_s, W_p,
                ts0, ts1, tp0, tp1):
    dv = dinv[...]
    s = _relu(dv * jnp.concatenate([Ss0[...], Ss1[...]], axis=1) + b_s[...])
    p = _relu(dv * jnp.concatenate([Sp0[...], Sp1[...]], axis=1) + b_p[...])
    hs = dv * jnp.dot(s, W_s[...], preferred_element_type=jnp.float32)
    hp = dv * jnp.dot(p, W_p[...], preferred_element_type=jnp.float32)
    ts0[...] = hs[:, :HH]
    ts1[...] = hs[:, HH:]
    tp0[...] = hp[:, :HH]
    tp1[...] = hp[:, HH:]


def _tc_round(Ss0, Ss1, Sp0, Sp1, dinv, b_s, b_p, W_s, W_p):
    return pl.pallas_call(
        _round_body,
        grid=(GRID,),
        in_specs=[_row_spec(HH)] * 4 + [_row_spec(1),
                  _full_spec((1, H)), _full_spec((1, H)),
                  _full_spec((H, H)), _full_spec((H, H))],
        out_specs=[_row_spec(HH)] * 4,
        out_shape=[jax.ShapeDtypeStruct((NP, HH), jnp.float32)] * 4,
    )(Ss0, Ss1, Sp0, Sp1, dinv, b_s, b_p, W_s, W_p)


def _mix_body(Ss0, Ss1, Sp0, Sp1, dinv, b_s, b_p, W_t, W_b, tc0, tc1):
    dv = dinv[...]
    s = _relu(dv * jnp.concatenate([Ss0[...], Ss1[...]], axis=1) + b_s[...])
    p = _relu(dv * jnp.concatenate([Sp0[...], Sp1[...]], axis=1) + b_p[...])
    hc = dv * (jnp.dot(s, W_t[...], preferred_element_type=jnp.float32)
               + jnp.dot(p, W_b[...], preferred_element_type=jnp.float32))
    tc0[...] = hc[:, :HH]
    tc1[...] = hc[:, HH:]


def _tc_mix(Ss0, Ss1, Sp0, Sp1, dinv, b_s, b_p, W_t, W_b):
    return pl.pallas_call(
        _mix_body,
        grid=(GRID,),
        in_specs=[_row_spec(HH)] * 4 + [_row_spec(1),
                  _full_spec((1, H)), _full_spec((1, H)),
                  _full_spec((H, H)), _full_spec((H, H))],
        out_specs=[_row_spec(HH)] * 2,
        out_shape=[jax.ShapeDtypeStruct((NP, HH), jnp.float32)] * 2,
    )(Ss0, Ss1, Sp0, Sp1, dinv, b_s, b_p, W_t, W_b)


def _fin_body(Sc0, Sc1, dinv, b_cc, W_d1, b_d1, W_d2, b_d2, W_d3, b_d3, out):
    dv = dinv[...]
    c = _relu(dv * jnp.concatenate([Sc0[...], Sc1[...]], axis=1) + b_cc[...])
    d = _relu(jnp.dot(c, W_d1[...], preferred_element_type=jnp.float32)
              + b_d1[...])
    d = _relu(jnp.dot(d, W_d2[...], preferred_element_type=jnp.float32)
              + b_d2[...])
    d3 = jnp.sum(d * W_d3[...], axis=1, keepdims=True) + b_d3[...]
    out[...] = 1.0 / (1.0 + jnp.exp(-d3))


def _tc_fin(Sc0, Sc1, dinv, b_cc, W_d1, b_d1, W_d2, b_d2, W_d3, b_d3):
    return pl.pallas_call(
        _fin_body,
        grid=(GRID,),
        in_specs=[_row_spec(HH)] * 2 + [_row_spec(1),
                  _full_spec((1, H)),
                  _full_spec((H, H)), _full_spec((1, H)),
                  _full_spec((H, HH)), _full_spec((1, HH)),
                  _full_spec((1, HH)), _full_spec((1, 1))],
        out_specs=[_row_spec(1)],
        out_shape=[jax.ShapeDtypeStruct((NP, 1), jnp.float32)],
    )(Sc0, Sc1, dinv, b_cc, W_d1, b_d1, W_d2, b_d2, W_d3, b_d3)[0]


@jax.jit
def _run(x, true_alpha_t, true_torque_t, edge_index,
         W_se, b_se, W_pe1, b_pe1, W_pe2, b_pe2,
         W_c1, b_c1, W_c2, b_c2, W_pc1, b_pc1, W_pc2, b_pc2,
         W_cc, b_cc, W_d1, b_d1, W_d2, b_d2, W_d3, b_d3):
    f32 = jnp.float32
    xp = jnp.zeros((NP, 2), f32).at[:N].set(x)
    pin = jnp.zeros((NP, 2), f32).at[:N, 0].set(true_alpha_t[:, 0])
    pin = pin.at[:N, 1].set(true_torque_t[:, 0])
    pad = jnp.full((EP - E,), N, jnp.int32)
    srcr = jnp.concatenate([edge_index[0], pad]).reshape(EROWS, 128)
    dstr = jnp.concatenate([edge_index[1], pad]).reshape(EROWS, 128)
    ones_t = jnp.ones((NP, HH), f32)

    def row(b):
        return b.reshape(1, -1)

    d0, _ = _sc_spmm(ones_t, ones_t, srcr, dstr)

    dinv, ts0, ts1, tp0, tp1 = _tc_enc(
        xp, pin, d0, W_se, row(b_se), W_pe1, row(b_pe1), W_pe2,
        row(b_pe2), W_c1, W_pc1)

    Ss0, Ss1 = _sc_spmm(ts0, ts1, srcr, dstr)
    Sp0, Sp1 = _sc_spmm(tp0, tp1, srcr, dstr)
    ts0, ts1, tp0, tp1 = _tc_round(Ss0, Ss1, Sp0, Sp1, dinv,
                                   row(b_c1), row(b_pc1), W_c2, W_pc2)

    Ss0, Ss1 = _sc_spmm(ts0, ts1, srcr, dstr)
    Sp0, Sp1 = _sc_spmm(tp0, tp1, srcr, dstr)
    tc0, tc1 = _tc_mix(Ss0, Ss1, Sp0, Sp1, dinv, row(b_c2), row(b_pc2),
                       W_cc[:H], W_cc[H:])

    Sc0, Sc1 = _sc_spmm(tc0, tc1, srcr, dstr)
    out = _tc_fin(Sc0, Sc1, dinv, row(b_cc), W_d1, row(b_d1),
                  W_d2, row(b_d2), W_d3.reshape(1, HH), b_d3.reshape(1, 1))
    return out[:N]


def kernel(x, true_alpha_t, true_torque_t, edge_index,
           W_se, b_se, W_pe1, b_pe1, W_pe2, b_pe2,
           W_c1, b_c1, W_c2, b_c2, W_pc1, b_pc1, W_pc2, b_pc2,
           W_cc, b_cc, W_d1, b_d1, W_d2, b_d2, W_d3, b_d3):
    return _run(x, true_alpha_t, true_torque_t, edge_index,
                W_se, b_se, W_pe1, b_pe1, W_pe2, b_pe2,
                W_c1, b_c1, W_c2, b_c2, W_pc1, b_pc1, W_pc2, b_pc2,
                W_cc, b_cc, W_d1, b_d1, W_d2, b_d2, W_d3, b_d3)


# speedup vs baseline: 10.6183x; 1.1833x over previous
import functools

import jax
import jax.numpy as jnp
from jax import lax
from jax.experimental import pallas as pl
from jax.experimental.pallas import tpu as pltpu
from jax.experimental.pallas import tpu_sc as plsc

N = 50000
E = 800000
H = 64
HH = 32

NP = 50176
EP = 819200
NTILES = 16
ROWS_PER_TILE = NP // NTILES
EROWS = EP // 128
EROWS_PER_TILE = EROWS // NTILES
C = 2
NC = EROWS_PER_TILE // C

BN = 512
GRID = NP // BN

_SC_MESH = plsc.VectorSubcoreMesh(core_axis_name="c", subcore_axis_name="s")
_SC_PARAMS = pltpu.CompilerParams(use_tc_tiling_on_sc=False)


def _spmm_body(t0, t1, idxc, out0, out1, idx, rows, acc, sem_i, sem_g, sem_s):
    c = lax.axis_index("c")
    s = lax.axis_index("s")
    rbase = s * ROWS_PER_TILE

    @pl.when(c == 0)
    def _():
        pltpu.sync_copy(t0.at[pl.ds(rbase, ROWS_PER_TILE)],
                        acc.at[pl.ds(rbase, ROWS_PER_TILE)])

    @pl.when(c == 1)
    def _():
        pltpu.sync_copy(t1.at[pl.ds(rbase, ROWS_PER_TILE)],
                        acc.at[pl.ds(rbase, ROWS_PER_TILE)])

    plsc.subcore_barrier()

    rbase2 = s * EROWS_PER_TILE

    def idx_load(k):
        m = lax.rem(k, 3)
        return pltpu.async_copy(idxc.at[pl.ds(rbase2 + k * C, C)],
                                idx.at[m], sem_i)

    def drain_idx():
        pltpu.make_async_copy(idxc.at[pl.ds(0, C)], idx.at[0], sem_i).wait()

    def fire_gather(k):
        m3 = lax.rem(k, 3)
        m2 = lax.rem(k, 2)

        @pl.when(c == 0)
        def _():
            for j in range(C):
                pltpu.async_copy(t0.at[idx.at[m3, j, 0]], rows.at[m2, j],
                                 sem_g)

        @pl.when(c == 1)
        def _():
            for j in range(C):
                pltpu.async_copy(t1.at[idx.at[m3, j, 0]], rows.at[m2, j],
                                 sem_g)

    def drain_gather():
        for j in range(C):
            pltpu.make_async_copy(t0.at[pl.ds(0, 128)], rows.at[0, j],
                                  sem_g).wait()

    def fire_scatter(k):
        m3 = lax.rem(k, 3)
        m2 = lax.rem(k, 2)
        for j in range(C):
            pltpu.async_copy(rows.at[m2, j], acc.at[idx.at[m3, j, 1]],
                             sem_s, add=True)

    def drain_scatter():
        for j in range(C):
            pltpu.make_async_copy(t0.at[pl.ds(0, 128)], rows.at[0, j],
                                  sem_s).wait()

    idx_load(0)
    drain_idx()
    idx_load(1)
    fire_gather(0)

    def step(k, carry):
        drain_gather()

        @pl.when(k >= 1)
        def _():
            drain_scatter()

        fire_scatter(k)

        @pl.when(k + 1 < NC)
        def _():
            drain_idx()

        @pl.when(k + 2 < NC)
        def _():
            idx_load(k + 2)

        @pl.when(k + 1 < NC)
        def _():
            fire_gather(k + 1)

        return carry

    lax.fori_loop(0, NC, step, 0)
    drain_scatter()
    plsc.subcore_barrier()

    @pl.when(c == 0)
    def _():
        pltpu.sync_copy(acc.at[pl.ds(rbase, ROWS_PER_TILE)],
                        out0.at[pl.ds(rbase, ROWS_PER_TILE)])

    @pl.when(c == 1)
    def _():
        pltpu.sync_copy(acc.at[pl.ds(rbase, ROWS_PER_TILE)],
                        out1.at[pl.ds(rbase, ROWS_PER_TILE)])


_sc_spmm = pl.kernel(
    _spmm_body,
    out_type=(jax.ShapeDtypeStruct((NP, HH), jnp.float32),
              jax.ShapeDtypeStruct((NP, HH), jnp.float32)),
    mesh=_SC_MESH,
    scratch_types=[
        pltpu.VMEM((3, C, 2, 128), jnp.int32),
        pltpu.VMEM((2, C, 128, HH), jnp.float32),
        pltpu.VMEM_SHARED((NP, HH), jnp.float32),
        pltpu.SemaphoreType.DMA,
        pltpu.SemaphoreType.DMA,
        pltpu.SemaphoreType.DMA,
    ],
    compiler_params=_SC_PARAMS,
)


def _row_spec(w):
    return pl.BlockSpec((BN, w), lambda i: (i, 0))


def _full_spec(shape):
    return pl.BlockSpec(shape, lambda i: tuple(0 for _ in shape))


def _relu(v):
    return jnp.maximum(v, 0.0)


def _enc_body(x, pin, d0, W_se, b_se, W_pe1, b_pe1, W_pe2, b_pe2,
              W_c1, W_pc1, dinv_o, ts0, ts1, tp0, tp1):
    dinv = lax.rsqrt(d0[:, :1])
    s0 = _relu(x[:, 0:1] * W_se[0:1, :] + x[:, 1:2] * W_se[1:2, :] + b_se[...])
    p = _relu(pin[:, 0:1] * W_pe1[0:1, :] + pin[:, 1:2] * W_pe1[1:2, :]
              + b_pe1[...])
    p0 = _relu(jnp.dot(p, W_pe2[...], preferred_element_type=jnp.float32)
               + b_pe2[...])
    hs = dinv * jnp.dot(s0, W_c1[...], preferred_element_type=jnp.float32)
    hp = dinv * jnp.dot(p0, W_pc1[...], preferred_element_type=jnp.float32)
    dinv_o[...] = dinv
    ts0[...] = hs[:, :HH]
    ts1[...] = hs[:, HH:]
    tp0[...] = hp[:, :HH]
    tp1[...] = hp[:, HH:]


def _tc_enc(x, pin, d0, W_se, b_se, W_pe1, b_pe1, W_pe2, b_pe2,
            W_c1, W_pc1):
    return pl.pallas_call(
        _enc_body,
        grid=(GRID,),
        in_specs=[_row_spec(2), _row_spec(2), _row_spec(HH),
                  _full_spec((2, H)), _full_spec((1, H)),
                  _full_spec((2, H)), _full_spec((1, H)),
                  _full_spec((H, H)), _full_spec((1, H)),
                  _full_spec((H, H)), _full_spec((H, H))],
        out_specs=[_row_spec(1), _row_spec(HH), _row_spec(HH),
                   _row_spec(HH), _row_spec(HH)],
        out_shape=[jax.ShapeDtypeStruct((NP, 1), jnp.float32)]
        + [jax.ShapeDtypeStruct((NP, HH), jnp.float32)] * 4,
    )(x, pin, d0, W_se, b_se, W_pe1, b_pe1, W_pe2, b_pe2, W_c1, W_pc1)


def _round_body(Ss0, Ss1, Sp0, Sp1, dinv, b_s, b_p, W_s, W_p,
                ts0, ts1, tp0, tp1):
    dv = dinv[...]
    s = _relu(dv * jnp.concatenate([Ss0[...], Ss1[...]], axis=1) + b_s[...])
    p = _relu(dv * jnp.concatenate([Sp0[...], Sp1[...]], axis=1) + b_p[...])
    hs = dv * jnp.dot(s, W_s[...], preferred_element_type=jnp.float32)
    hp = dv * jnp.dot(p, W_p[...], preferred_element_type=jnp.float32)
    ts0[...] = hs[:, :HH]
    ts1[...] = hs[:, HH:]
    tp0[...] = hp[:, :HH]
    tp1[...] = hp[:, HH:]


def _tc_round(Ss0, Ss1, Sp0, Sp1, dinv, b_s, b_p, W_s, W_p):
    return pl.pallas_call(
        _round_body,
        grid=(GRID,),
        in_specs=[_row_spec(HH)] * 4 + [_row_spec(1),
                  _full_spec((1, H)), _full_spec((1, H)),
                  _full_spec((H, H)), _full_spec((H, H))],
        out_specs=[_row_spec(HH)] * 4,
        out_shape=[jax.ShapeDtypeStruct((NP, HH), jnp.float32)] * 4,
    )(Ss0, Ss1, Sp0, Sp1, dinv, b_s, b_p, W_s, W_p)


def _mix_body(Ss0, Ss1, Sp0, Sp1, dinv, b_s, b_p, W_t, W_b, tc0, tc1):
    dv = dinv[...]
    s = _relu(dv * jnp.concatenate([Ss0[...], Ss1[...]], axis=1) + b_s[...])
    p = _relu(dv * jnp.concatenate([Sp0[...], Sp1[...]], axis=1) + b_p[...])
    hc = dv * (jnp.dot(s, W_t[...], preferred_element_type=jnp.float32)
               + jnp.dot(p, W_b[...], preferred_element_type=jnp.float32))
    tc0[...] = hc[:, :HH]
    tc1[...] = hc[:, HH:]


def _tc_mix(Ss0, Ss1, Sp0, Sp1, dinv, b_s, b_p, W_t, W_b):
    return pl.pallas_call(
        _mix_body,
        grid=(GRID,),
        in_specs=[_row_spec(HH)] * 4 + [_row_spec(1),
                  _full_spec((1, H)), _full_spec((1, H)),
                  _full_spec((H, H)), _full_spec((H, H))],
        out_specs=[_row_spec(HH)] * 2,
        out_shape=[jax.ShapeDtypeStruct((NP, HH), jnp.float32)] * 2,
    )(Ss0, Ss1, Sp0, Sp1, dinv, b_s, b_p, W_t, W_b)


def _fin_body(Sc0, Sc1, dinv, b_cc, W_d1, b_d1, W_d2, b_d2, W_d3, b_d3, out):
    dv = dinv[...]
    c = _relu(dv * jnp.concatenate([Sc0[...], Sc1[...]], axis=1) + b_cc[...])
    d = _relu(jnp.dot(c, W_d1[...], preferred_element_type=jnp.float32)
              + b_d1[...])
    d = _relu(jnp.dot(d, W_d2[...], preferred_element_type=jnp.float32)
              + b_d2[...])
    d3 = jnp.sum(d * W_d3[...], axis=1, keepdims=True) + b_d3[...]
    out[...] = 1.0 / (1.0 + jnp.exp(-d3))


def _tc_fin(Sc0, Sc1, dinv, b_cc, W_d1, b_d1, W_d2, b_d2, W_d3, b_d3):
    return pl.pallas_call(
        _fin_body,
        grid=(GRID,),
        in_specs=[_row_spec(HH)] * 2 + [_row_spec(1),
                  _full_spec((1, H)),
                  _full_spec((H, H)), _full_spec((1, H)),
                  _full_spec((H, HH)), _full_spec((1, HH)),
                  _full_spec((1, HH)), _full_spec((1, 1))],
        out_specs=[_row_spec(1)],
        out_shape=[jax.ShapeDtypeStruct((NP, 1), jnp.float32)],
    )(Sc0, Sc1, dinv, b_cc, W_d1, b_d1, W_d2, b_d2, W_d3, b_d3)[0]


@jax.jit
def _run(x, true_alpha_t, true_torque_t, edge_index,
         W_se, b_se, W_pe1, b_pe1, W_pe2, b_pe2,
         W_c1, b_c1, W_c2, b_c2, W_pc1, b_pc1, W_pc2, b_pc2,
         W_cc, b_cc, W_d1, b_d1, W_d2, b_d2, W_d3, b_d3):
    f32 = jnp.float32
    xp = jnp.zeros((NP, 2), f32).at[:N].set(x)
    pin = jnp.zeros((NP, 2), f32).at[:N, 0].set(true_alpha_t[:, 0])
    pin = pin.at[:N, 1].set(true_torque_t[:, 0])
    pad = jnp.full((EP - E,), N, jnp.int32)
    srcr = jnp.concatenate([edge_index[0], pad]).reshape(EROWS, 128)
    dstr = jnp.concatenate([edge_index[1], pad]).reshape(EROWS, 128)
    idxc = jnp.stack([srcr, dstr], axis=1)
    ones_t = jnp.ones((NP, HH), f32)

    def row(b):
        return b.reshape(1, -1)

    d0, _ = _sc_spmm(ones_t, ones_t, idxc)

    dinv, ts0, ts1, tp0, tp1 = _tc_enc(
        xp, pin, d0, W_se, row(b_se), W_pe1, row(b_pe1), W_pe2,
        row(b_pe2), W_c1, W_pc1)

    Ss0, Ss1 = _sc_spmm(ts0, ts1, idxc)
    Sp0, Sp1 = _sc_spmm(tp0, tp1, idxc)
    ts0, ts1, tp0, tp1 = _tc_round(Ss0, Ss1, Sp0, Sp1, dinv,
                                   row(b_c1), row(b_pc1), W_c2, W_pc2)

    Ss0, Ss1 = _sc_spmm(ts0, ts1, idxc)
    Sp0, Sp1 = _sc_spmm(tp0, tp1, idxc)
    tc0, tc1 = _tc_mix(Ss0, Ss1, Sp0, Sp1, dinv, row(b_c2), row(b_pc2),
                       W_cc[:H], W_cc[H:])

    Sc0, Sc1 = _sc_spmm(tc0, tc1, idxc)
    out = _tc_fin(Sc0, Sc1, dinv, row(b_cc), W_d1, row(b_d1),
                  W_d2, row(b_d2), W_d3.reshape(1, HH), b_d3.reshape(1, 1))
    return out[:N]


def kernel(x, true_alpha_t, true_torque_t, edge_index,
           W_se, b_se, W_pe1, b_pe1, W_pe2, b_pe2,
           W_c1, b_c1, W_c2, b_c2, W_pc1, b_pc1, W_pc2, b_pc2,
           W_cc, b_cc, W_d1, b_d1, W_d2, b_d2, W_d3, b_d3):
    return _run(x, true_alpha_t, true_torque_t, edge_index,
                W_se, b_se, W_pe1, b_pe1, W_pe2, b_pe2,
                W_c1, b_c1, W_c2, b_c2, W_pc1, b_pc1, W_pc2, b_pc2,
                W_cc, b_cc, W_d1, b_d1, W_d2, b_d2, W_d3, b_d3)


# speedup vs baseline: 11.7263x; 1.1043x over previous
import functools

import jax
import jax.numpy as jnp
from jax import lax
from jax.experimental import pallas as pl
from jax.experimental.pallas import tpu as pltpu
from jax.experimental.pallas import tpu_sc as plsc

N = 50000
E = 800000
H = 64
HH = 32

NP = 50176
EP = 819200
NTILES = 16
ROWS_PER_TILE = NP // NTILES
EROWS = EP // 128
EROWS_PER_TILE = EROWS // NTILES
C = 2
NC = EROWS_PER_TILE // C

BN = 512
GRID = NP // BN

_SC_MESH = plsc.VectorSubcoreMesh(core_axis_name="c", subcore_axis_name="s")
_SC_PARAMS = pltpu.CompilerParams(use_tc_tiling_on_sc=False)


def _spmm_body(t0, t1, idxc, out0, out1, idx, rows, acc,
               sem_i0, sem_i1, sem_g0, sem_g1, sem_s):
    c = lax.axis_index("c")
    s = lax.axis_index("s")
    rbase = s * ROWS_PER_TILE

    @pl.when(c == 0)
    def _():
        pltpu.sync_copy(t0.at[pl.ds(rbase, ROWS_PER_TILE)],
                        acc.at[pl.ds(rbase, ROWS_PER_TILE)])

    @pl.when(c == 1)
    def _():
        pltpu.sync_copy(t1.at[pl.ds(rbase, ROWS_PER_TILE)],
                        acc.at[pl.ds(rbase, ROWS_PER_TILE)])

    plsc.subcore_barrier()

    rbase2 = s * EROWS_PER_TILE

    def idx_load(k):
        sem = [sem_i0, sem_i1][k % 2] if isinstance(k, int) else None
        m = lax.rem(k, 4) if not isinstance(k, int) else (k % 4)
        if sem is None:
            raise AssertionError
        pltpu.async_copy(idxc.at[pl.ds(rbase2 + k * C, C)], idx.at[m], sem)

    def idx_load_dyn(k, sem):
        pltpu.async_copy(idxc.at[pl.ds(rbase2 + k * C, C)],
                         idx.at[lax.rem(k, 4)], sem)

    def drain_idx(sem):
        pltpu.make_async_copy(idxc.at[pl.ds(0, C)], idx.at[0], sem).wait()

    def fire_gather(k, sem):
        m4 = lax.rem(k, 4) if not isinstance(k, int) else (k % 4)
        m3 = lax.rem(k, 3) if not isinstance(k, int) else (k % 3)

        @pl.when(c == 0)
        def _():
            for j in range(C):
                pltpu.async_copy(t0.at[idx.at[m4, j, 0]], rows.at[m3, j], sem)

        @pl.when(c == 1)
        def _():
            for j in range(C):
                pltpu.async_copy(t1.at[idx.at[m4, j, 0]], rows.at[m3, j], sem)

    def drain_gather(sem):
        for j in range(C):
            pltpu.make_async_copy(t0.at[pl.ds(0, 128)], rows.at[0, j],
                                  sem).wait()

    def fire_scatter(k):
        m4 = lax.rem(k, 4) if not isinstance(k, int) else (k % 4)
        m3 = lax.rem(k, 3) if not isinstance(k, int) else (k % 3)
        for j in range(C):
            pltpu.async_copy(rows.at[m3, j], acc.at[idx.at[m4, j, 1]],
                             sem_s, add=True)

    def drain_scatter():
        for j in range(C):
            pltpu.make_async_copy(t0.at[pl.ds(0, 128)], rows.at[0, j],
                                  sem_s).wait()

    idx_load(0)
    drain_idx(sem_i0)
    idx_load(1)
    drain_idx(sem_i1)
    fire_gather(0, sem_g0)
    idx_load(2)
    fire_gather(1, sem_g1)

    def halfstep(k, sg_k, sg_k2, si_k, si_k1):
        drain_gather(sg_k)

        @pl.when(k >= 1)
        def _():
            drain_scatter()

        fire_scatter(k)

        @pl.when(k + 2 < NC)
        def _():
            drain_idx(si_k)

        @pl.when(k + 3 < NC)
        def _():
            idx_load_dyn(k + 3, si_k1)

        @pl.when(k + 2 < NC)
        def _():
            fire_gather(k + 2, sg_k)

    def step(i, carry):
        k = i * 2
        halfstep(k, sem_g0, sem_g1, sem_i0, sem_i1)
        halfstep(k + 1, sem_g1, sem_g0, sem_i1, sem_i0)
        return carry

    lax.fori_loop(0, NC // 2, step, 0)
    drain_scatter()
    plsc.subcore_barrier()

    @pl.when(c == 0)
    def _():
        pltpu.sync_copy(acc.at[pl.ds(rbase, ROWS_PER_TILE)],
                        out0.at[pl.ds(rbase, ROWS_PER_TILE)])

    @pl.when(c == 1)
    def _():
        pltpu.sync_copy(acc.at[pl.ds(rbase, ROWS_PER_TILE)],
                        out1.at[pl.ds(rbase, ROWS_PER_TILE)])


_sc_spmm = pl.kernel(
    _spmm_body,
    out_type=(jax.ShapeDtypeStruct((NP, HH), jnp.float32),
              jax.ShapeDtypeStruct((NP, HH), jnp.float32)),
    mesh=_SC_MESH,
    scratch_types=[
        pltpu.VMEM((4, C, 2, 128), jnp.int32),
        pltpu.VMEM((3, C, 128, HH), jnp.float32),
        pltpu.VMEM_SHARED((NP, HH), jnp.float32),
        pltpu.SemaphoreType.DMA,
        pltpu.SemaphoreType.DMA,
        pltpu.SemaphoreType.DMA,
        pltpu.SemaphoreType.DMA,
        pltpu.SemaphoreType.DMA,
    ],
    compiler_params=_SC_PARAMS,
)


def _row_spec(w):
    return pl.BlockSpec((BN, w), lambda i: (i, 0))


def _full_spec(shape):
    return pl.BlockSpec(shape, lambda i: tuple(0 for _ in shape))


def _relu(v):
    return jnp.maximum(v, 0.0)


def _enc_body(x, pin, d0, W_se, b_se, W_pe1, b_pe1, W_pe2, b_pe2,
              W_c1, W_pc1, dinv_o, ts0, ts1, tp0, tp1):
    dinv = lax.rsqrt(d0[:, :1])
    s0 = _relu(x[:, 0:1] * W_se[0:1, :] + x[:, 1:2] * W_se[1:2, :] + b_se[...])
    p = _relu(pin[:, 0:1] * W_pe1[0:1, :] + pin[:, 1:2] * W_pe1[1:2, :]
              + b_pe1[...])
    p0 = _relu(jnp.dot(p, W_pe2[...], preferred_element_type=jnp.float32)
               + b_pe2[...])
    hs = dinv * jnp.dot(s0, W_c1[...], preferred_element_type=jnp.float32)
    hp = dinv * jnp.dot(p0, W_pc1[...], preferred_element_type=jnp.float32)
    dinv_o[...] = dinv
    ts0[...] = hs[:, :HH]
    ts1[...] = hs[:, HH:]
    tp0[...] = hp[:, :HH]
    tp1[...] = hp[:, HH:]


def _tc_enc(x, pin, d0, W_se, b_se, W_pe1, b_pe1, W_pe2, b_pe2,
            W_c1, W_pc1):
    return pl.pallas_call(
        _enc_body,
        grid=(GRID,),
        in_specs=[_row_spec(2), _row_spec(2), _row_spec(HH),
                  _full_spec((2, H)), _full_spec((1, H)),
                  _full_spec((2, H)), _full_spec((1, H)),
                  _full_spec((H, H)), _full_spec((1, H)),
                  _full_spec((H, H)), _full_spec((H, H))],
        out_specs=[_row_spec(1), _row_spec(HH), _row_spec(HH),
                   _row_spec(HH), _row_spec(HH)],
        out_shape=[jax.ShapeDtypeStruct((NP, 1), jnp.float32)]
        + [jax.ShapeDtypeStruct((NP, HH), jnp.float32)] * 4,
    )(x, pin, d0, W_se, b_se, W_pe1, b_pe1, W_pe2, b_pe2, W_c1, W_pc1)


def _round_body(Ss0, Ss1, Sp0, Sp1, dinv, b_s, b_p, W_s, W_p,
                ts0, ts1, tp0, tp1):
    dv = dinv[...]
    s = _relu(dv * jnp.concatenate([Ss0[...], Ss1[...]], axis=1) + b_s[...])
    p = _relu(dv * jnp.concatenate([Sp0[...], Sp1[...]], axis=1) + b_p[...])
    hs = dv * jnp.dot(s, W_s[...], preferred_element_type=jnp.float32)
    hp = dv * jnp.dot(p, W_p[...], preferred_element_type=jnp.float32)
    ts0[...] = hs[:, :HH]
    ts1[...] = hs[:, HH:]
    tp0[...] = hp[:, :HH]
    tp1[...] = hp[:, HH:]


def _tc_round(Ss0, Ss1, Sp0, Sp1, dinv, b_s, b_p, W_s, W_p):
    return pl.pallas_call(
        _round_body,
        grid=(GRID,),
        in_specs=[_row_spec(HH)] * 4 + [_row_spec(1),
                  _full_spec((1, H)), _full_spec((1, H)),
                  _full_spec((H, H)), _full_spec((H, H))],
        out_specs=[_row_spec(HH)] * 4,
        out_shape=[jax.ShapeDtypeStruct((NP, HH), jnp.float32)] * 4,
    )(Ss0, Ss1, Sp0, Sp1, dinv, b_s, b_p, W_s, W_p)


def _mix_body(Ss0, Ss1, Sp0, Sp1, dinv, b_s, b_p, W_t, W_b, tc0, tc1):
    dv = dinv[...]
    s = _relu(dv * jnp.concatenate([Ss0[...], Ss1[...]], axis=1) + b_s[...])
    p = _relu(dv * jnp.concatenate([Sp0[...], Sp1[...]], axis=1) + b_p[...])
    hc = dv * (jnp.dot(s, W_t[...], preferred_element_type=jnp.float32)
               + jnp.dot(p, W_b[...], preferred_element_type=jnp.float32))
    tc0[...] = hc[:, :HH]
    tc1[...] = hc[:, HH:]


def _tc_mix(Ss0, Ss1, Sp0, Sp1, dinv, b_s, b_p, W_t, W_b):
    return pl.pallas_call(
        _mix_body,
        grid=(GRID,),
        in_specs=[_row_spec(HH)] * 4 + [_row_spec(1),
                  _full_spec((1, H)), _full_spec((1, H)),
                  _full_spec((H, H)), _full_spec((H, H))],
        out_specs=[_row_spec(HH)] * 2,
        out_shape=[jax.ShapeDtypeStruct((NP, HH), jnp.float32)] * 2,
    )(Ss0, Ss1, Sp0, Sp1, dinv, b_s, b_p, W_t, W_b)


def _fin_body(Sc0, Sc1, dinv, b_cc, W_d1, b_d1, W_d2, b_d2, W_d3, b_d3, out):
    dv = dinv[...]
    c = _relu(dv * jnp.concatenate([Sc0[...], Sc1[...]], axis=1) + b_cc[...])
    d = _relu(jnp.dot(c, W_d1[...], preferred_element_type=jnp.float32)
              + b_d1[...])
    d = _relu(jnp.dot(d, W_d2[...], preferred_element_type=jnp.float32)
              + b_d2[...])
    d3 = jnp.sum(d * W_d3[...], axis=1, keepdims=True) + b_d3[...]
    out[...] = 1.0 / (1.0 + jnp.exp(-d3))


def _tc_fin(Sc0, Sc1, dinv, b_cc, W_d1, b_d1, W_d2, b_d2, W_d3, b_d3):
    return pl.pallas_call(
        _fin_body,
        grid=(GRID,),
        in_specs=[_row_spec(HH)] * 2 + [_row_spec(1),
                  _full_spec((1, H)),
                  _full_spec((H, H)), _full_spec((1, H)),
                  _full_spec((H, HH)), _full_spec((1, HH)),
                  _full_spec((1, HH)), _full_spec((1, 1))],
        out_specs=[_row_spec(1)],
        out_shape=[jax.ShapeDtypeStruct((NP, 1), jnp.float32)],
    )(Sc0, Sc1, dinv, b_cc, W_d1, b_d1, W_d2, b_d2, W_d3, b_d3)[0]


@jax.jit
def _run(x, true_alpha_t, true_torque_t, edge_index,
         W_se, b_se, W_pe1, b_pe1, W_pe2, b_pe2,
         W_c1, b_c1, W_c2, b_c2, W_pc1, b_pc1, W_pc2, b_pc2,
         W_cc, b_cc, W_d1, b_d1, W_d2, b_d2, W_d3, b_d3):
    f32 = jnp.float32
    xp = jnp.zeros((NP, 2), f32).at[:N].set(x)
    pin = jnp.zeros((NP, 2), f32).at[:N, 0].set(true_alpha_t[:, 0])
    pin = pin.at[:N, 1].set(true_torque_t[:, 0])
    pad = jnp.full((EP - E,), N, jnp.int32)
    srcr = jnp.concatenate([edge_index[0], pad]).reshape(EROWS, 128)
    dstr = jnp.concatenate([edge_index[1], pad]).reshape(EROWS, 128)
    idxc = jnp.stack([srcr, dstr], axis=1)
    ones_t = jnp.ones((NP, HH), f32)

    def row(b):
        return b.reshape(1, -1)

    d0, _ = _sc_spmm(ones_t, ones_t, idxc)

    dinv, ts0, ts1, tp0, tp1 = _tc_enc(
        xp, pin, d0, W_se, row(b_se), W_pe1, row(b_pe1), W_pe2,
        row(b_pe2), W_c1, W_pc1)

    Ss0, Ss1 = _sc_spmm(ts0, ts1, idxc)
    Sp0, Sp1 = _sc_spmm(tp0, tp1, idxc)
    ts0, ts1, tp0, tp1 = _tc_round(Ss0, Ss1, Sp0, Sp1, dinv,
                                   row(b_c1), row(b_pc1), W_c2, W_pc2)

    Ss0, Ss1 = _sc_spmm(ts0, ts1, idxc)
    Sp0, Sp1 = _sc_spmm(tp0, tp1, idxc)
    tc0, tc1 = _tc_mix(Ss0, Ss1, Sp0, Sp1, dinv, row(b_c2), row(b_pc2),
                       W_cc[:H], W_cc[H:])

    Sc0, Sc1 = _sc_spmm(tc0, tc1, idxc)
    out = _tc_fin(Sc0, Sc1, dinv, row(b_cc), W_d1, row(b_d1),
                  W_d2, row(b_d2), W_d3.reshape(1, HH), b_d3.reshape(1, 1))
    return out[:N]


def kernel(x, true_alpha_t, true_torque_t, edge_index,
           W_se, b_se, W_pe1, b_pe1, W_pe2, b_pe2,
           W_c1, b_c1, W_c2, b_c2, W_pc1, b_pc1, W_pc2, b_pc2,
           W_cc, b_cc, W_d1, b_d1, W_d2, b_d2, W_d3, b_d3):
    return _run(x, true_alpha_t, true_torque_t, edge_index,
                W_se, b_se, W_pe1, b_pe1, W_pe2, b_pe2,
                W_c1, b_c1, W_c2, b_c2, W_pc1, b_pc1, W_pc2, b_pc2,
                W_cc, b_cc, W_d1, b_d1, W_d2, b_d2, W_d3, b_d3)
